# baseline (speedup 1.0000x reference)
import functools
import jax
import jax.numpy as jnp
from jax import lax
from jax.experimental import pallas as pl
from jax.experimental.pallas import tpu as pltpu, tpu_sc as plsc

V = 10000
E = 160000
LN = 128
S = 32
C = 40
MU = 0.9
CPAD = 128
BV = 1000
NC = 2
NS = 16
NW = NC * NS
EW = E // NW
K = 40
NCH = EW // K
EW2 = E // NS
NCH2 = EW2 // K
SCALE = MU / S

_f32 = jnp.float32


def _dense_kernel(fm_ref, w1_ref, w2_ref, wr_ref, bx_ref, br_ref,
                  p1_ref, p2_ref, tq_ref):
    f = fm_ref[...]
    p1_ref[...] = 2.0 * jnp.dot(f, w1_ref[...], preferred_element_type=_f32) + bx_ref[0:1, :]
    p2_ref[...] = 2.0 * jnp.dot(f, w2_ref[...], preferred_element_type=_f32)
    tq = jnp.tanh(jnp.dot(f, wr_ref[...], preferred_element_type=_f32) + br_ref[0:1, :])
    sb = jnp.sum(tq, axis=1, keepdims=True)
    tq_ref[...] = jnp.concatenate(
        [tq, sb, jnp.zeros((tq.shape[0], 95), _f32)], axis=1)


def _edge_body(p1_hbm, p2_hbm, tqx_hbm, xnode_hbm, xneis_hbm, dg_hbm, y_hbm, c_hbm,
               ytab, ctab, r1, r2, bb, idxu, idxp, idxn, idxr, idxc, dgv, yv, cv, zb,
               sem1, sem2, sem3):
    c = lax.axis_index("c")
    s = lax.axis_index("s")
    vbase = c * 5000

    dgv[pl.ds(K, 16)] = jnp.ones((16,), _f32)
    idxp[pl.ds(K, 16)] = jnp.zeros((16,), jnp.int32)

    def yvfill(i, _):
        for t in range(6):
            yv[i, pl.ds(32 + 16 * t, 16)] = jnp.zeros((16,), _f32)
        return 0
    lax.fori_loop(0, K, yvfill, 0)

    def zfill(i, _):
        for t in range(8):
            zb[i, pl.ds(16 * t, 16)] = jnp.zeros((16,), _f32)
        return 0
    lax.fori_loop(0, 80, zfill, 0)

    @pl.when(s < NS - 1)
    def _():
        pltpu.sync_copy(zb, ytab.at[pl.ds(pl.multiple_of(s * 80, 8), 80)])

    @pl.when(s == NS - 1)
    def _():
        pltpu.sync_copy(zb.at[pl.ds(0, 56)], ytab.at[pl.ds(1200, 56)])

    @pl.when(s == 0)
    def _():
        pltpu.sync_copy(zb.at[pl.ds(0, 48)], ctab)
    plsc.subcore_barrier()

    def chunk(ci, _):
        base = pl.multiple_of(s * EW2 + ci * K, 8)
        pltpu.sync_copy(xnode_hbm.at[pl.ds(base, K)], idxu)
        pltpu.sync_copy(xnode_hbm.at[pl.ds(base, K)], idxp.at[pl.ds(0, K)])
        pltpu.sync_copy(xneis_hbm.at[pl.ds(base, K)], idxn)
        pltpu.sync_copy(dg_hbm.at[pl.ds(base, K)], dgv.at[pl.ds(0, K)])
        for st in (0, 16, 24):
            uvv = idxu[pl.ds(st, 16)]
            loc = uvv - vbase
            inb = (loc >= 0) & (loc < 5000)
            safe = jnp.where(inb, loc, 5000)
            idxr[pl.ds(st, 16)] = lax.shift_right_logical(safe, 2)
            idxc[pl.ds(st, 16)] = jnp.where(inb, lax.shift_right_logical(loc, 7),
                                            jnp.full((16,), 40, jnp.int32))
        cp1 = pltpu.async_copy(p1_hbm.at[idxu], r1, sem1)
        cp2 = pltpu.async_copy(p2_hbm.at[idxn], r2, sem2)
        cp3 = pltpu.async_copy(tqx_hbm.at[idxu], bb, sem3)
        cp1.wait()
        cp2.wait()
        cp3.wait()

        def edge(k, _):
            uv = idxp[pl.ds(k, 16)]
            locv = uv - vbase
            ul = locv[0]

            @pl.when((ul >= 0) & (ul < 5000))
            def _():
                b0 = bb[k, 0:16]
                b1 = bb[k, 16:32]
                bs = bb[k, 32:48]
                dv = dgv[pl.ds(k, 16)]
                skv = SCALE / dv
                sk = skv[0]
                offv = (locv & 3) * 32
                off = offv[0]
                acc0 = jnp.zeros((16,), _f32)
                acc1 = jnp.zeros((16,), _f32)
                for j in range(S):
                    bj = b0[j] if j < 16 else b1[j - 16]
                    v0 = 2 * j
                    pre0 = r1[k, pl.ds(v0 * 16, 16)] + r2[k, pl.ds(v0 * 16, 16)]
                    acc0 = acc0 + bj / (jnp.exp(pre0) + 1.0)
                    pre1 = r1[k, pl.ds(v0 * 16 + 16, 16)] + r2[k, pl.ds(v0 * 16 + 16, 16)]
                    acc1 = acc1 + bj / (jnp.exp(pre1) + 1.0)
                sb = bs[0]
                for t in range(8):
                    yv[k, pl.ds(16 * t, 16)] = jnp.zeros((16,), _f32)
                    cv[k, pl.ds(16 * t, 16)] = jnp.zeros((16,), _f32)
                yv[k, pl.ds(off, 16)] = (sb - 2.0 * acc0) * sk
                yv[k, pl.ds(off + 16, 16)] = (sb - 2.0 * acc1) * sk
                colv = locv & 127
                coffv = (colv >> 4) * 16
                lanev = colv & 15
                lane = lax.iota(jnp.int32, 16)
                onehot = jnp.where(lane == lanev[0], 1.0, 0.0).astype(_f32)
                cv[k, pl.ds(coffv[0], 16)] = onehot
            return 0
        lax.fori_loop(0, K, edge, 0)
        pltpu.sync_copy(yv, ytab.at[idxr], add=True)
        pltpu.sync_copy(cv, ctab.at[idxc], add=True)
        return 0
    lax.fori_loop(0, NCH2, chunk, 0)
    plsc.subcore_barrier()

    @pl.when(s < NS - 1)
    def _():
        soff = pl.multiple_of(s * 80, 8)
        pltpu.sync_copy(ytab.at[pl.ds(soff, 80)],
                        y_hbm.at[pl.ds(c * 1256 + soff, 80)])

    @pl.when(s == NS - 1)
    def _():
        pltpu.sync_copy(ytab.at[pl.ds(1200, 56)],
                        y_hbm.at[pl.ds(c * 1256 + 1200, 56)])

    @pl.when(s == 0)
    def _():
        pltpu.sync_copy(ctab, c_hbm.at[pl.ds(c * 48, 48)])


def _head_kernel(y_ref, tqx_ref, cnt_ref, w_ref, bias_ref, out_ref):
    h2 = cnt_ref[:, 0:1] * (y_ref[...] + tqx_ref[:, :S])
    logits = jnp.dot(h2, w_ref[...], preferred_element_type=_f32)
    logits = logits + bias_ref[0:1, :]
    m = jnp.max(logits, axis=-1, keepdims=True)
    lse = jnp.log(jnp.sum(jnp.exp(logits - m), axis=-1, keepdims=True)) + m
    out_ref[...] = logits - lse


def kernel(feat_Matrix, X_Node, X_Neis, dg_list, W_xi, b_xi, W_rou, b_rou, W_out, b_out):
    idx = jnp.arange(S * S)
    perm = (idx % S) * S + idx // S
    W1T = W_xi[perm, :LN].T
    W2T = W_xi[perm, LN:].T
    bx2 = jnp.broadcast_to(2.0 * b_xi[perm], (8, S * S))
    WrT = W_rou.T
    br = jnp.broadcast_to(b_rou, (8, S))

    P1c, P2c, tqx = pl.pallas_call(
        _dense_kernel,
        grid=(V // BV,),
        in_specs=[
            pl.BlockSpec((BV, LN), lambda i: (i, 0)),
            pl.BlockSpec((LN, S * S), lambda i: (0, 0)),
            pl.BlockSpec((LN, S * S), lambda i: (0, 0)),
            pl.BlockSpec((LN, S), lambda i: (0, 0)),
            pl.BlockSpec((8, S * S), lambda i: (0, 0)),
            pl.BlockSpec((8, S), lambda i: (0, 0)),
        ],
        out_specs=[
            pl.BlockSpec((BV, S * S), lambda i: (i, 0)),
            pl.BlockSpec((BV, S * S), lambda i: (i, 0)),
            pl.BlockSpec((BV, 128), lambda i: (i, 0)),
        ],
        out_shape=[
            jax.ShapeDtypeStruct((V, S * S), _f32),
            jax.ShapeDtypeStruct((V, S * S), _f32),
            jax.ShapeDtypeStruct((V, 128), _f32),
        ],
    )(feat_Matrix, W1T, W2T, WrT, bx2, br)

    mesh = plsc.VectorSubcoreMesh(core_axis_name="c", subcore_axis_name="s")
    edge_k = functools.partial(
        pl.kernel,
        out_type=(jax.ShapeDtypeStruct((NC * 1256, 128), _f32),
                  jax.ShapeDtypeStruct((NC * 48, 128), _f32)),
        mesh=mesh,
        scratch_types=[
            pltpu.VMEM_SHARED((1256, 128), _f32),
            pltpu.VMEM_SHARED((48, 128), _f32),
            pltpu.VMEM((K, S * S), _f32),
            pltpu.VMEM((K, S * S), _f32),
            pltpu.VMEM((K, 128), _f32),
            pltpu.VMEM((K,), jnp.int32),
            pltpu.VMEM((K + 16,), jnp.int32),
            pltpu.VMEM((K,), jnp.int32),
            pltpu.VMEM((K,), jnp.int32),
            pltpu.VMEM((K,), jnp.int32),
            pltpu.VMEM((K + 16,), _f32),
            pltpu.VMEM((K, 128), _f32),
            pltpu.VMEM((K, 128), _f32),
            pltpu.VMEM((80, 128), _f32),
            pltpu.SemaphoreType.DMA,
            pltpu.SemaphoreType.DMA,
            pltpu.SemaphoreType.DMA,
        ],
    )(_edge_body)
    yflat, chbm = edge_k(P1c, P2c, tqx, X_Node, X_Neis, dg_list)
    z0 = yflat[0:1250, :].reshape(5000, S)
    z1 = yflat[1256:2506, :].reshape(5000, S)
    zz = jnp.concatenate([z0, z1], axis=0)
    cnt0 = chbm[0:40, :].reshape(-1)[0:5000]
    cnt1 = chbm[48:88, :].reshape(-1)[0:5000]
    cnt = jnp.concatenate([cnt0, cnt1], axis=0)
    cntx = jnp.broadcast_to(cnt[:, None], (V, 8))

    Wp = jnp.zeros((S, CPAD), _f32).at[:, :C].set(W_out.T)
    bp = jnp.full((CPAD,), -1e30, _f32).at[:C].set(b_out)
    bp = jnp.broadcast_to(bp, (8, CPAD))

    out = pl.pallas_call(
        _head_kernel,
        grid=(V // BV,),
        in_specs=[
            pl.BlockSpec((BV, S), lambda i: (i, 0)),
            pl.BlockSpec((BV, 128), lambda i: (i, 0)),
            pl.BlockSpec((BV, 8), lambda i: (i, 0)),
            pl.BlockSpec((S, CPAD), lambda i: (0, 0)),
            pl.BlockSpec((8, CPAD), lambda i: (0, 0)),
        ],
        out_specs=pl.BlockSpec((BV, CPAD), lambda i: (i, 0)),
        out_shape=jax.ShapeDtypeStruct((V, CPAD), _f32),
    )(zz, tqx, cntx, Wp, bp)
    return out[:, :C]


# speedup vs baseline: 1.0513x; 1.0513x over previous
import functools
import jax
import jax.numpy as jnp
from jax import lax
from jax.experimental import pallas as pl
from jax.experimental.pallas import tpu as pltpu, tpu_sc as plsc

V = 10000
E = 160000
LN = 128
S = 32
C = 40
MU = 0.9
CPAD = 128
BV = 1000
NC = 2
NS = 16
NW = NC * NS
EW = E // NW
K = 40
NCH = EW // K
EW2 = E // NS
BLK = 1000
SCALE = MU / S

_f32 = jnp.float32


def _dense_kernel(fm_ref, w1_ref, w2_ref, wr_ref, bx_ref, br_ref,
                  p1_ref, p2_ref, tq_ref):
    f = fm_ref[...]
    p1_ref[...] = 2.0 * jnp.dot(f, w1_ref[...], preferred_element_type=_f32) + bx_ref[0:1, :]
    p2_ref[...] = 2.0 * jnp.dot(f, w2_ref[...], preferred_element_type=_f32)
    tq = jnp.tanh(jnp.dot(f, wr_ref[...], preferred_element_type=_f32) + br_ref[0:1, :])
    sb = jnp.sum(tq, axis=1, keepdims=True)
    tq_ref[...] = jnp.concatenate(
        [tq, sb, jnp.zeros((tq.shape[0], 95), _f32)], axis=1)


def _edge_body(p1_hbm, p2_hbm, tqx_hbm, xnode_hbm, xneis_hbm, dg_hbm, y_hbm, c_hbm,
               ytab, ctab, r1, r2, bb, ublk, nblk, dblk, idxr, idxid, ctloc, yv, zb,
               sem1, sem2, sem3):
    c = lax.axis_index("c")
    s = lax.axis_index("s")
    vbase = c * 5000

    dblk[pl.ds(BLK, 16)] = jnp.ones((16,), _f32)
    ublk[pl.ds(BLK, 16)] = jnp.zeros((16,), jnp.int32)
    for t in (0, 16, 32):
        idxid[pl.ds(t, 16)] = lax.iota(jnp.int32, 16) + t

    def yvfill(i, _):
        for t in range(6):
            yv[i, pl.ds(32 + 16 * t, 16)] = jnp.zeros((16,), _f32)
        return 0
    lax.fori_loop(0, K, yvfill, 0)

    def ctfill(i, _):
        for t in range(8):
            ctloc[i, pl.ds(16 * t, 16)] = jnp.zeros((16,), _f32)
        return 0
    lax.fori_loop(0, 48, ctfill, 0)

    def zfill(i, _):
        for t in range(8):
            zb[i, pl.ds(16 * t, 16)] = jnp.zeros((16,), _f32)
        return 0
    lax.fori_loop(0, 80, zfill, 0)

    @pl.when(s < NS - 1)
    def _():
        pltpu.sync_copy(zb, ytab.at[pl.ds(pl.multiple_of(s * 80, 8), 80)])

    @pl.when(s == NS - 1)
    def _():
        pltpu.sync_copy(zb.at[pl.ds(0, 56)], ytab.at[pl.ds(1200, 56)])

    @pl.when(s == 0)
    def _():
        pltpu.sync_copy(zb.at[pl.ds(0, 48)], ctab)
    plsc.subcore_barrier()

    def block(b, _):
        bbase = pl.multiple_of(s * EW2 + b * BLK, 8)
        pltpu.sync_copy(xnode_hbm.at[pl.ds(bbase, BLK)], ublk.at[pl.ds(0, BLK)])
        pltpu.sync_copy(xneis_hbm.at[pl.ds(bbase, BLK)], nblk)
        pltpu.sync_copy(dg_hbm.at[pl.ds(bbase, BLK)], dblk.at[pl.ds(0, BLK)])

        def chunk(ci, _):
            co = ci * K
            for st in (0, 16, 24):
                uvv = ublk[pl.ds(co + st, 16)]
                loc = uvv - vbase
                inb = (loc >= 0) & (loc < 5000)
                safe = jnp.where(inb, loc, 5000)
                idxr[pl.ds(st, 16)] = lax.shift_right_logical(safe, 2)
            cp1 = pltpu.async_copy(p1_hbm.at[ublk.at[pl.ds(co, K)]], r1, sem1)
            cp2 = pltpu.async_copy(p2_hbm.at[nblk.at[pl.ds(co, K)]], r2, sem2)
            cp3 = pltpu.async_copy(tqx_hbm.at[ublk.at[pl.ds(co, K)]], bb, sem3)
            cp1.wait()
            cp2.wait()
            cp3.wait()

            def edge(k, _):
                uv = ublk[pl.ds(co + k, 16)]
                locv = uv - vbase
                ul = locv[0]

                @pl.when((ul >= 0) & (ul < 5000))
                def _():
                    b0 = bb[k, 0:16]
                    b1 = bb[k, 16:32]
                    bs = bb[k, 32:48]
                    dv = dblk[pl.ds(co + k, 16)]
                    skv = SCALE / dv
                    sk = skv[0]
                    offv = (locv & 3) * 32
                    off = offv[0]
                    acc0 = jnp.zeros((16,), _f32)
                    acc1 = jnp.zeros((16,), _f32)
                    for j in range(S):
                        bj = b0[j] if j < 16 else b1[j - 16]
                        v0 = 2 * j
                        pre0 = r1[k, pl.ds(v0 * 16, 16)] + r2[k, pl.ds(v0 * 16, 16)]
                        acc0 = acc0 + bj / (jnp.exp(pre0) + 1.0)
                        pre1 = r1[k, pl.ds(v0 * 16 + 16, 16)] + r2[k, pl.ds(v0 * 16 + 16, 16)]
                        acc1 = acc1 + bj / (jnp.exp(pre1) + 1.0)
                    sb = bs[0]
                    for t in range(8):
                        yv[k, pl.ds(16 * t, 16)] = jnp.zeros((16,), _f32)
                    yv[k, pl.ds(off, 16)] = (sb - 2.0 * acc0) * sk
                    yv[k, pl.ds(off + 16, 16)] = (sb - 2.0 * acc1) * sk
                    crv = lax.shift_right_logical(locv, 7)
                    crow = crv[0]
                    colv = locv & 127
                    coffv = (colv >> 4) * 16
                    coff = coffv[0]
                    lanev = colv & 15
                    lane = lax.iota(jnp.int32, 16)
                    onehot = jnp.where(lane == lanev[0], 1.0, 0.0).astype(_f32)
                    told = ctloc[crow, pl.ds(coff, 16)]
                    ctloc[crow, pl.ds(coff, 16)] = told + onehot
                return 0
            lax.fori_loop(0, K, edge, 0)
            pltpu.sync_copy(yv, ytab.at[idxr], add=True)
            return 0
        lax.fori_loop(0, BLK // K, chunk, 0)
        return 0
    lax.fori_loop(0, EW2 // BLK, block, 0)
    pltpu.sync_copy(ctloc, ctab.at[idxid], add=True)
    plsc.subcore_barrier()

    @pl.when(s < NS - 1)
    def _():
        soff = pl.multiple_of(s * 80, 8)
        pltpu.sync_copy(ytab.at[pl.ds(soff, 80)],
                        y_hbm.at[pl.ds(c * 1256 + soff, 80)])

    @pl.when(s == NS - 1)
    def _():
        pltpu.sync_copy(ytab.at[pl.ds(1200, 56)],
                        y_hbm.at[pl.ds(c * 1256 + 1200, 56)])

    @pl.when(s == 0)
    def _():
        pltpu.sync_copy(ctab, c_hbm.at[pl.ds(c * 48, 48)])


def _head_kernel(y_ref, tqx_ref, cnt_ref, w_ref, bias_ref, out_ref):
    h2 = cnt_ref[:, 0:1] * (y_ref[...] + tqx_ref[:, :S])
    logits = jnp.dot(h2, w_ref[...], preferred_element_type=_f32)
    logits = logits + bias_ref[0:1, :]
    m = jnp.max(logits, axis=-1, keepdims=True)
    lse = jnp.log(jnp.sum(jnp.exp(logits - m), axis=-1, keepdims=True)) + m
    out_ref[...] = logits - lse


def kernel(feat_Matrix, X_Node, X_Neis, dg_list, W_xi, b_xi, W_rou, b_rou, W_out, b_out):
    idx = jnp.arange(S * S)
    perm = (idx % S) * S + idx // S
    W1T = W_xi[perm, :LN].T
    W2T = W_xi[perm, LN:].T
    bx2 = jnp.broadcast_to(2.0 * b_xi[perm], (8, S * S))
    WrT = W_rou.T
    br = jnp.broadcast_to(b_rou, (8, S))

    P1c, P2c, tqx = pl.pallas_call(
        _dense_kernel,
        grid=(V // BV,),
        in_specs=[
            pl.BlockSpec((BV, LN), lambda i: (i, 0)),
            pl.BlockSpec((LN, S * S), lambda i: (0, 0)),
            pl.BlockSpec((LN, S * S), lambda i: (0, 0)),
            pl.BlockSpec((LN, S), lambda i: (0, 0)),
            pl.BlockSpec((8, S * S), lambda i: (0, 0)),
            pl.BlockSpec((8, S), lambda i: (0, 0)),
        ],
        out_specs=[
            pl.BlockSpec((BV, S * S), lambda i: (i, 0)),
            pl.BlockSpec((BV, S * S), lambda i: (i, 0)),
            pl.BlockSpec((BV, 128), lambda i: (i, 0)),
        ],
        out_shape=[
            jax.ShapeDtypeStruct((V, S * S), _f32),
            jax.ShapeDtypeStruct((V, S * S), _f32),
            jax.ShapeDtypeStruct((V, 128), _f32),
        ],
    )(feat_Matrix, W1T, W2T, WrT, bx2, br)

    mesh = plsc.VectorSubcoreMesh(core_axis_name="c", subcore_axis_name="s")
    edge_k = functools.partial(
        pl.kernel,
        out_type=(jax.ShapeDtypeStruct((NC * 1256, 128), _f32),
                  jax.ShapeDtypeStruct((NC * 48, 128), _f32)),
        mesh=mesh,
        scratch_types=[
            pltpu.VMEM_SHARED((1256, 128), _f32),
            pltpu.VMEM_SHARED((48, 128), _f32),
            pltpu.VMEM((K, S * S), _f32),
            pltpu.VMEM((K, S * S), _f32),
            pltpu.VMEM((K, 128), _f32),
            pltpu.VMEM((BLK + 16,), jnp.int32),
            pltpu.VMEM((BLK,), jnp.int32),
            pltpu.VMEM((BLK + 16,), _f32),
            pltpu.VMEM((K,), jnp.int32),
            pltpu.VMEM((48,), jnp.int32),
            pltpu.VMEM((48, 128), _f32),
            pltpu.VMEM((K, 128), _f32),
            pltpu.VMEM((80, 128), _f32),
            pltpu.SemaphoreType.DMA,
            pltpu.SemaphoreType.DMA,
            pltpu.SemaphoreType.DMA,
        ],
    )(_edge_body)
    yflat, chbm = edge_k(P1c, P2c, tqx, X_Node, X_Neis, dg_list)
    z0 = yflat[0:1250, :].reshape(5000, S)
    z1 = yflat[1256:2506, :].reshape(5000, S)
    zz = jnp.concatenate([z0, z1], axis=0)
    cnt0 = chbm[0:40, :].reshape(-1)[0:5000]
    cnt1 = chbm[48:88, :].reshape(-1)[0:5000]
    cnt = jnp.concatenate([cnt0, cnt1], axis=0)
    cntx = jnp.broadcast_to(cnt[:, None], (V, 8))

    Wp = jnp.zeros((S, CPAD), _f32).at[:, :C].set(W_out.T)
    bp = jnp.full((CPAD,), -1e30, _f32).at[:C].set(b_out)
    bp = jnp.broadcast_to(bp, (8, CPAD))

    out = pl.pallas_call(
        _head_kernel,
        grid=(V // BV,),
        in_specs=[
            pl.BlockSpec((BV, S), lambda i: (i, 0)),
            pl.BlockSpec((BV, 128), lambda i: (i, 0)),
            pl.BlockSpec((BV, 8), lambda i: (i, 0)),
            pl.BlockSpec((S, CPAD), lambda i: (0, 0)),
            pl.BlockSpec((8, CPAD), lambda i: (0, 0)),
        ],
        out_specs=pl.BlockSpec((BV, CPAD), lambda i: (i, 0)),
        out_shape=jax.ShapeDtypeStruct((V, CPAD), _f32),
    )(zz, tqx, cntx, Wp, bp)
    return out[:, :C]


# speedup vs baseline: 1.1880x; 1.1300x over previous
import functools
import jax
import jax.numpy as jnp
from jax import lax
from jax.experimental import pallas as pl
from jax.experimental.pallas import tpu as pltpu, tpu_sc as plsc

V = 10000
E = 160000
LN = 128
S = 32
C = 40
MU = 0.9
CPAD = 128
BV = 1000
NC = 2
NS = 16
NW = NC * NS
EW = E // NW
K = 40
NCH = EW // K
EW2 = E // NS
BLK = 1000
SCALE = MU / S

_f32 = jnp.float32

TB = 4.2
TC0 = 0.9984870022967067
TC1 = -0.31612427400549187
TC2 = 0.0980005857919904
TC3 = -0.021586585574683046
TC4 = 0.003101555064339166
TC5 = -0.0002801223621391636
TC6 = 1.5237073426592907e-05
TC7 = -4.545474746910083e-07
TC8 = 5.70137986821706e-09


def _dense_kernel(fm_ref, w1_ref, w2_ref, wr_ref, bx_ref, br_ref,
                  p1_ref, p2_ref, tq_ref):
    f = fm_ref[...]
    p1_ref[...] = jnp.dot(f, w1_ref[...], preferred_element_type=_f32) + bx_ref[0:1, :]
    p2_ref[...] = jnp.dot(f, w2_ref[...], preferred_element_type=_f32)
    tq = jnp.tanh(jnp.dot(f, wr_ref[...], preferred_element_type=_f32) + br_ref[0:1, :])
    sb = jnp.sum(tq, axis=1, keepdims=True)
    tq_ref[...] = jnp.concatenate(
        [tq, sb, jnp.zeros((tq.shape[0], 95), _f32)], axis=1)


def _edge_body(p1_hbm, p2_hbm, tqx_hbm, xnode_hbm, xneis_hbm, dg_hbm, y_hbm, c_hbm,
               ytab, ctab, r1, r2, bb, ublk, nblk, dblk, idxr, idxid, ctloc, yv, zb,
               sem1, sem2, sem3):
    c = lax.axis_index("c")
    s = lax.axis_index("s")
    vbase = c * 5000

    dblk[pl.ds(BLK, 16)] = jnp.ones((16,), _f32)
    ublk[pl.ds(BLK, 16)] = jnp.zeros((16,), jnp.int32)
    for t in (0, 16, 32):
        idxid[pl.ds(t, 16)] = lax.iota(jnp.int32, 16) + t

    def yvfill(i, _):
        for t in range(6):
            yv[i, pl.ds(32 + 16 * t, 16)] = jnp.zeros((16,), _f32)
        return 0
    lax.fori_loop(0, K, yvfill, 0)

    def ctfill(i, _):
        for t in range(8):
            ctloc[i, pl.ds(16 * t, 16)] = jnp.zeros((16,), _f32)
        return 0
    lax.fori_loop(0, 48, ctfill, 0)

    def zfill(i, _):
        for t in range(8):
            zb[i, pl.ds(16 * t, 16)] = jnp.zeros((16,), _f32)
        return 0
    lax.fori_loop(0, 80, zfill, 0)

    @pl.when(s < NS - 1)
    def _():
        pltpu.sync_copy(zb, ytab.at[pl.ds(pl.multiple_of(s * 80, 8), 80)])

    @pl.when(s == NS - 1)
    def _():
        pltpu.sync_copy(zb.at[pl.ds(0, 56)], ytab.at[pl.ds(1200, 56)])

    @pl.when(s == 0)
    def _():
        pltpu.sync_copy(zb.at[pl.ds(0, 48)], ctab)
    plsc.subcore_barrier()

    def block(b, _):
        bbase = pl.multiple_of(s * EW2 + b * BLK, 8)
        pltpu.sync_copy(xnode_hbm.at[pl.ds(bbase, BLK)], ublk.at[pl.ds(0, BLK)])
        pltpu.sync_copy(xneis_hbm.at[pl.ds(bbase, BLK)], nblk)
        pltpu.sync_copy(dg_hbm.at[pl.ds(bbase, BLK)], dblk.at[pl.ds(0, BLK)])

        def chunk(ci, _):
            co = ci * K
            for st in (0, 16, 24):
                uvv = ublk[pl.ds(co + st, 16)]
                loc = uvv - vbase
                inb = (loc >= 0) & (loc < 5000)
                safe = jnp.where(inb, loc, 5000)
                idxr[pl.ds(st, 16)] = lax.shift_right_logical(safe, 2)
            cp1 = pltpu.async_copy(p1_hbm.at[ublk.at[pl.ds(co, K)]], r1, sem1)
            cp2 = pltpu.async_copy(p2_hbm.at[nblk.at[pl.ds(co, K)]], r2, sem2)
            cp3 = pltpu.async_copy(tqx_hbm.at[ublk.at[pl.ds(co, K)]], bb, sem3)
            cp1.wait()
            cp2.wait()
            cp3.wait()

            def edge(k, _):
                uv = ublk[pl.ds(co + k, 16)]
                locv = uv - vbase
                ul = locv[0]

                @pl.when((ul >= 0) & (ul < 5000))
                def _():
                    b0 = bb[k, 0:16]
                    b1 = bb[k, 16:32]
                    dv = dblk[pl.ds(co + k, 16)]
                    skv = SCALE / dv
                    sk = skv[0]
                    offv = (locv & 3) * 32
                    off = offv[0]
                    acc0 = jnp.zeros((16,), _f32)
                    acc1 = jnp.zeros((16,), _f32)

                    def _ptanh(x):
                        xc = jnp.minimum(jnp.maximum(x, -TB), TB)
                        y2 = xc * xc
                        p = TC8
                        for cc in (TC7, TC6, TC5, TC4, TC3, TC2, TC1, TC0):
                            p = p * y2 + cc
                        return xc * p

                    for j in range(S):
                        bj = b0[j] if j < 16 else b1[j - 16]
                        v0 = 2 * j
                        pre0 = r1[k, pl.ds(v0 * 16, 16)] + r2[k, pl.ds(v0 * 16, 16)]
                        acc0 = acc0 + bj * _ptanh(pre0)
                        pre1 = r1[k, pl.ds(v0 * 16 + 16, 16)] + r2[k, pl.ds(v0 * 16 + 16, 16)]
                        acc1 = acc1 + bj * _ptanh(pre1)
                    for t in range(8):
                        yv[k, pl.ds(16 * t, 16)] = jnp.zeros((16,), _f32)
                    yv[k, pl.ds(off, 16)] = acc0 * sk
                    yv[k, pl.ds(off + 16, 16)] = acc1 * sk
                    crv = lax.shift_right_logical(locv, 7)
                    crow = crv[0]
                    colv = locv & 127
                    coffv = (colv >> 4) * 16
                    coff = coffv[0]
                    lanev = colv & 15
                    lane = lax.iota(jnp.int32, 16)
                    onehot = jnp.where(lane == lanev[0], 1.0, 0.0).astype(_f32)
                    told = ctloc[crow, pl.ds(coff, 16)]
                    ctloc[crow, pl.ds(coff, 16)] = told + onehot
                return 0
            lax.fori_loop(0, K, edge, 0)
            pltpu.sync_copy(yv, ytab.at[idxr], add=True)
            return 0
        lax.fori_loop(0, BLK // K, chunk, 0)
        return 0
    lax.fori_loop(0, EW2 // BLK, block, 0)
    pltpu.sync_copy(ctloc, ctab.at[idxid], add=True)
    plsc.subcore_barrier()

    @pl.when(s < NS - 1)
    def _():
        soff = pl.multiple_of(s * 80, 8)
        pltpu.sync_copy(ytab.at[pl.ds(soff, 80)],
                        y_hbm.at[pl.ds(c * 1256 + soff, 80)])

    @pl.when(s == NS - 1)
    def _():
        pltpu.sync_copy(ytab.at[pl.ds(1200, 56)],
                        y_hbm.at[pl.ds(c * 1256 + 1200, 56)])

    @pl.when(s == 0)
    def _():
        pltpu.sync_copy(ctab, c_hbm.at[pl.ds(c * 48, 48)])


def _head_kernel(y_ref, tqx_ref, cnt_ref, w_ref, bias_ref, out_ref):
    h2 = cnt_ref[:, 0:1] * (y_ref[...] + tqx_ref[:, :S])
    logits = jnp.dot(h2, w_ref[...], preferred_element_type=_f32)
    logits = logits + bias_ref[0:1, :]
    m = jnp.max(logits, axis=-1, keepdims=True)
    lse = jnp.log(jnp.sum(jnp.exp(logits - m), axis=-1, keepdims=True)) + m
    out_ref[...] = logits - lse


def kernel(feat_Matrix, X_Node, X_Neis, dg_list, W_xi, b_xi, W_rou, b_rou, W_out, b_out):
    idx = jnp.arange(S * S)
    perm = (idx % S) * S + idx // S
    W1T = W_xi[perm, :LN].T
    W2T = W_xi[perm, LN:].T
    bx2 = jnp.broadcast_to(b_xi[perm], (8, S * S))
    WrT = W_rou.T
    br = jnp.broadcast_to(b_rou, (8, S))

    P1c, P2c, tqx = pl.pallas_call(
        _dense_kernel,
        grid=(V // BV,),
        in_specs=[
            pl.BlockSpec((BV, LN), lambda i: (i, 0)),
            pl.BlockSpec((LN, S * S), lambda i: (0, 0)),
            pl.BlockSpec((LN, S * S), lambda i: (0, 0)),
            pl.BlockSpec((LN, S), lambda i: (0, 0)),
            pl.BlockSpec((8, S * S), lambda i: (0, 0)),
            pl.BlockSpec((8, S), lambda i: (0, 0)),
        ],
        out_specs=[
            pl.BlockSpec((BV, S * S), lambda i: (i, 0)),
            pl.BlockSpec((BV, S * S), lambda i: (i, 0)),
            pl.BlockSpec((BV, 128), lambda i: (i, 0)),
        ],
        out_shape=[
            jax.ShapeDtypeStruct((V, S * S), _f32),
            jax.ShapeDtypeStruct((V, S * S), _f32),
            jax.ShapeDtypeStruct((V, 128), _f32),
        ],
    )(feat_Matrix, W1T, W2T, WrT, bx2, br)

    mesh = plsc.VectorSubcoreMesh(core_axis_name="c", subcore_axis_name="s")
    edge_k = functools.partial(
        pl.kernel,
        out_type=(jax.ShapeDtypeStruct((NC * 1256, 128), _f32),
                  jax.ShapeDtypeStruct((NC * 48, 128), _f32)),
        mesh=mesh,
        scratch_types=[
            pltpu.VMEM_SHARED((1256, 128), _f32),
            pltpu.VMEM_SHARED((48, 128), _f32),
            pltpu.VMEM((K, S * S), _f32),
            pltpu.VMEM((K, S * S), _f32),
            pltpu.VMEM((K, 128), _f32),
            pltpu.VMEM((BLK + 16,), jnp.int32),
            pltpu.VMEM((BLK,), jnp.int32),
            pltpu.VMEM((BLK + 16,), _f32),
            pltpu.VMEM((K,), jnp.int32),
            pltpu.VMEM((48,), jnp.int32),
            pltpu.VMEM((48, 128), _f32),
            pltpu.VMEM((K, 128), _f32),
            pltpu.VMEM((80, 128), _f32),
            pltpu.SemaphoreType.DMA,
            pltpu.SemaphoreType.DMA,
            pltpu.SemaphoreType.DMA,
        ],
    )(_edge_body)
    yflat, chbm = edge_k(P1c, P2c, tqx, X_Node, X_Neis, dg_list)
    z0 = yflat[0:1250, :].reshape(5000, S)
    z1 = yflat[1256:2506, :].reshape(5000, S)
    zz = jnp.concatenate([z0, z1], axis=0)
    cnt0 = chbm[0:40, :].reshape(-1)[0:5000]
    cnt1 = chbm[48:88, :].reshape(-1)[0:5000]
    cnt = jnp.concatenate([cnt0, cnt1], axis=0)
    cntx = jnp.broadcast_to(cnt[:, None], (V, 8))

    Wp = jnp.zeros((S, CPAD), _f32).at[:, :C].set(W_out.T)
    bp = jnp.full((CPAD,), -1e30, _f32).at[:C].set(b_out)
    bp = jnp.broadcast_to(bp, (8, CPAD))

    out = pl.pallas_call(
        _head_kernel,
        grid=(V // BV,),
        in_specs=[
            pl.BlockSpec((BV, S), lambda i: (i, 0)),
            pl.BlockSpec((BV, 128), lambda i: (i, 0)),
            pl.BlockSpec((BV, 8), lambda i: (i, 0)),
            pl.BlockSpec((S, CPAD), lambda i: (0, 0)),
            pl.BlockSpec((8, CPAD), lambda i: (0, 0)),
        ],
        out_specs=pl.BlockSpec((BV, CPAD), lambda i: (i, 0)),
        out_shape=jax.ShapeDtypeStruct((V, CPAD), _f32),
    )(zz, tqx, cntx, Wp, bp)
    return out[:, :C]
